# Initial kernel scaffold; baseline (speedup 1.0000x reference)
import jax
import jax.numpy as jnp
from jax.experimental import pallas as pl


def kernel(x):
    raise NotImplementedError("write your pallas kernel here")


# speedup vs baseline: 46.6948x; 46.6948x over previous
import jax
import jax.numpy as jnp
from jax.experimental import pallas as pl

N = 576
H = 384
BW = 32
K_ZERO = int(N * N * 30 / 100)


def _body(x_ref, out_ref):
    xb = x_ref[0]
    g = jnp.mean(xb, axis=1, keepdims=True)
    m = jnp.max(xb, axis=1, keepdims=True)
    adj = jax.lax.dot_general(
        g, m, (((1,), (1,)), ((), ())), preferred_element_type=jnp.float32,
        precision=jax.lax.Precision.HIGHEST,
    )
    b = jax.lax.bitcast_convert_type(adj, jnp.int32)
    key = jnp.where(b >= 0, b, jnp.int32(-2147483648) - b)
    key = jax.lax.shift_right_arithmetic(key, 1)

    def step(_, carry):
        lo, hi = carry
        mid = jax.lax.shift_right_arithmetic(lo + hi, 1)
        c = jnp.sum((key <= mid).astype(jnp.int32))
        pred = c >= K_ZERO + 1
        return (jnp.where(pred, lo, mid), jnp.where(pred, mid, hi))

    lo0 = jnp.int32(-(2**30) - 1)
    hi0 = jnp.int32(2**30)
    _, thr = jax.lax.fori_loop(0, 32, step, (lo0, hi0))
    out_ref[0] = jnp.where(key >= thr, jax.nn.sigmoid(adj), 0.0)


def kernel(x):
    b, w, n, h = x.shape
    xr = x.reshape(b * w, n, h)
    dmap = pl.pallas_call(
        _body,
        grid=(BW,),
        in_specs=[pl.BlockSpec((1, N, H), lambda i: (i, 0, 0))],
        out_specs=pl.BlockSpec((1, N, N), lambda i: (i, 0, 0)),
        out_shape=jax.ShapeDtypeStruct((BW, N, N), jnp.float32),
    )(xr)
    return xr, dmap


# speedup vs baseline: 160.7490x; 3.4425x over previous
import jax
import jax.numpy as jnp
from jax import lax
from jax.experimental import pallas as pl
from jax.experimental.pallas import tpu as pltpu
from jax.experimental.pallas import tpu_sc as plsc

N = 576
H = 384
BW = 32
K_ZERO = int(N * N * 30 / 100)
NPAD = 1024
NVREG = NPAD // 16
NJ = N // 16
INT_MIN32 = -2147483648
KEY_INF = 1069547520


def _stage_a_body(x_ref, g_ref, m_ref):
    xb = x_ref[0]
    g_ref[0] = jnp.mean(xb, axis=1, keepdims=True)
    m_ref[0] = jnp.max(xb, axis=1, keepdims=True)


def _keys31(adj):
    b = lax.bitcast_convert_type(adj, jnp.int32)
    key = jnp.where(b >= 0, b, jnp.int32(INT_MIN32) - b)
    return lax.shift_right_arithmetic(key, 1)


def _stage_b_body(g_ref, m_ref, thr_ref, out_ref):
    g = g_ref[0]
    m = m_ref[0]
    adj = lax.dot_general(
        g, m, (((1,), (1,)), ((), ())), preferred_element_type=jnp.float32,
        precision=lax.Precision.HIGHEST,
    )
    thr = thr_ref[0, 0, 0]
    out_ref[0] = jnp.where(_keys31(adj) >= thr, jax.nn.sigmoid(adj), 0.0)


def _sorted16(y):
    out = plsc.sort_key_val(y, y)
    return out[0] if isinstance(out, (tuple, list)) else out


def _sc_body(g_hbm, m_hbm, thr_hbm, gs_v, m_v, out_v):
    wid = lax.axis_index("s") * 2 + lax.axis_index("c")
    pltpu.sync_copy(g_hbm.at[wid], gs_v.at[pl.ds(0, N)])
    pltpu.sync_copy(m_hbm.at[wid], m_v)

    inf16 = jnp.full((16,), jnp.inf, jnp.float32)

    def pad_body(i, c):
        gs_v[pl.ds(N + i * 16, 16)] = inf16
        return c

    lax.fori_loop(0, (NPAD - N) // 16, pad_body, 0)

    one16 = jnp.full((16,), 1.0, jnp.float32)
    neg16 = jnp.full((16,), -1.0, jnp.float32)

    def vsort_pass(kv):
        def body(v, c):
            vec = gs_v[pl.ds(v * 16, 16)]
            asc = (v & kv) == 0
            s = jnp.where(asc, one16, neg16)
            gs_v[pl.ds(v * 16, 16)] = _sorted16(vec * s) * s
            return c

        lax.fori_loop(0, NVREG, body, 0)

    vsort_pass(1)
    for t in range(6):
        kv = 2 << t
        for u in range(t, -1, -1):
            jv = 1 << u

            def cross_body(p, c, u=u, jv=jv, kv=kv):
                a = ((p >> u) << (u + 1)) | (p & (jv - 1))
                b = a + jv
                asc = (a & kv) == 0
                va = gs_v[pl.ds(a * 16, 16)]
                vb = gs_v[pl.ds(b * 16, 16)]
                lo = jnp.minimum(va, vb)
                hi = jnp.maximum(va, vb)
                gs_v[pl.ds(a * 16, 16)] = jnp.where(asc, lo, hi)
                gs_v[pl.ds(b * 16, 16)] = jnp.where(asc, hi, lo)
                return c

            lax.fori_loop(0, NVREG // 2, cross_body, 0)
        vsort_pass(kv)

    zero16 = jnp.zeros((16,), jnp.int32)
    n16 = jnp.full((16,), N, jnp.int32)
    int_min16 = jnp.full((16,), INT_MIN32, jnp.int32)
    pinf16 = jnp.full((16,), jnp.inf, jnp.float32)

    def count_quad(j4, tot, v):
        cnts = []
        for q in range(4):
            mvec = m_v[pl.ds((j4 * 4 + q) * 16, 16)]
            neg = mvec < 0.0
            loi = zero16
            hii = n16
            for _ in range(10):
                midi = lax.shift_right_arithmetic(loi + hii, 1)
                gv = plsc.load_gather(gs_v, [midi])
                le = (gv * mvec) <= v
                pr = le != neg
                loi = jnp.where(pr, midi + 1, loi)
                hii = jnp.where(pr, hii, midi)
            cnts.append(jnp.where(neg, n16 - loi, loi))
        return tot + jnp.sum(cnts[0] + cnts[1] + cnts[2] + cnts[3])

    def titer(_, carry):
        lo_k, hi_k = carry
        mid = lax.shift_right_arithmetic(lo_k + hi_k, 1)
        midv = jnp.full((16,), mid, jnp.int32)
        bp = lax.shift_left(midv, 1) | 1
        bits = jnp.where(bp >= 0, bp, int_min16 - bp)
        v = plsc.bitcast(bits, jnp.float32)
        v = jnp.where(midv >= KEY_INF, pinf16, v)
        c = lax.fori_loop(0, NJ // 4, lambda j4, tot: count_quad(j4, tot, v),
                          jnp.int32(0))
        pred = c >= K_ZERO + 1
        return (jnp.where(pred, lo_k, mid), jnp.where(pred, mid, hi_k))

    _, thr = lax.fori_loop(
        0, 32, titer, (jnp.int32(-(2**30) - 1), jnp.int32(2**30)))
    out_v[...] = jnp.full((16,), thr, jnp.int32)
    pltpu.sync_copy(out_v, thr_hbm.at[wid])


def _thresholds_sc(g2, m2):
    mesh = plsc.VectorSubcoreMesh(
        core_axis_name="c", subcore_axis_name="s", num_cores=2,
        num_subcores=16)
    return pl.kernel(
        _sc_body,
        out_type=jax.ShapeDtypeStruct((BW, 16), jnp.int32),
        mesh=mesh,
        scratch_types=[
            pltpu.VMEM((NPAD,), jnp.float32),
            pltpu.VMEM((N,), jnp.float32),
            pltpu.VMEM((16,), jnp.int32),
        ],
        compiler_params=pltpu.CompilerParams(
            needs_layout_passes=False, use_tc_tiling_on_sc=False),
    )(g2, m2)


def kernel(x):
    b, w, n, h = x.shape
    xr = x.reshape(b * w, n, h)
    g3, m3 = pl.pallas_call(
        _stage_a_body,
        grid=(BW,),
        in_specs=[pl.BlockSpec((1, N, H), lambda i: (i, 0, 0))],
        out_specs=[
            pl.BlockSpec((1, N, 1), lambda i: (i, 0, 0)),
            pl.BlockSpec((1, N, 1), lambda i: (i, 0, 0)),
        ],
        out_shape=[
            jax.ShapeDtypeStruct((BW, N, 1), jnp.float32),
            jax.ShapeDtypeStruct((BW, N, 1), jnp.float32),
        ],
    )(xr)
    thr = _thresholds_sc(g3.reshape(BW, N), m3.reshape(BW, N))
    dmap = pl.pallas_call(
        _stage_b_body,
        grid=(BW,),
        in_specs=[
            pl.BlockSpec((1, N, 1), lambda i: (i, 0, 0)),
            pl.BlockSpec((1, N, 1), lambda i: (i, 0, 0)),
            pl.BlockSpec((1, 1, 16), lambda i: (i, 0, 0),
                         memory_space=pltpu.SMEM),
        ],
        out_specs=pl.BlockSpec((1, N, N), lambda i: (i, 0, 0)),
        out_shape=jax.ShapeDtypeStruct((BW, N, N), jnp.float32),
    )(g3, m3, thr.reshape(BW, 1, 16))
    return xr, dmap


# speedup vs baseline: 191.0965x; 1.1888x over previous
import jax
import jax.numpy as jnp
from jax import lax
from jax.experimental import pallas as pl
from jax.experimental.pallas import tpu as pltpu
from jax.experimental.pallas import tpu_sc as plsc

N = 576
H = 384
BW = 32
K_ZERO = int(N * N * 30 / 100)
NPAD = 1024
NVREG = NPAD // 16
NJ = N // 16
INT_MIN32 = -2147483648
KEY_INF = 1069547520

_DOTDIM_T = (((1,), (1,)), ((), ()))


def _stage_a_body(x_ref, g_ref, m_ref):
    xb = x_ref[0]
    gc = jnp.mean(xb, axis=1, keepdims=True)
    mc = jnp.max(xb, axis=1, keepdims=True)
    ones11 = jnp.ones((1, 1), jnp.float32)
    g_ref[0] = lax.dot_general(ones11, gc, _DOTDIM_T,
                               preferred_element_type=jnp.float32,
                               precision=lax.Precision.HIGHEST)
    m_ref[0] = lax.dot_general(ones11, mc, _DOTDIM_T,
                               preferred_element_type=jnp.float32,
                               precision=lax.Precision.HIGHEST)


def _stage_b_body(g_ref, m_ref, w_ref, out_ref):
    g_row = g_ref[0]
    m_row = m_ref[0]
    ones11 = jnp.ones((1, 1), jnp.float32)
    g_col = lax.dot_general(g_row, ones11, (((0,), (0,)), ((), ())),
                            preferred_element_type=jnp.float32,
                            precision=lax.Precision.HIGHEST)
    adj = g_col * m_row
    w = w_ref[0, 0, 0]
    out_ref[0] = jnp.where(adj >= w, jax.nn.sigmoid(adj), 0.0)


def _sorted16(y):
    out = plsc.sort_key_val(y, y)
    return out[0] if isinstance(out, (tuple, list)) else out


def _keys31_v(f):
    b = plsc.bitcast(f, jnp.int32)
    key = jnp.where(b >= 0, b, jnp.full((16,), INT_MIN32, jnp.int32) - b)
    return lax.shift_right_arithmetic(key, 1)


def _decode31_hi(midv, int_min16, pinf16):
    bp = lax.shift_left(midv, 1) | 1
    bits = jnp.where(bp >= 0, bp, int_min16 - bp)
    v = plsc.bitcast(bits, jnp.float32)
    return jnp.where(midv >= KEY_INF, pinf16, v)


def _sc_body(g_hbm, m_hbm, thr_hbm, gs_v, m_v, out_v):
    wid = lax.axis_index("s") * 2 + lax.axis_index("c")
    pltpu.sync_copy(g_hbm.at[wid], gs_v.at[pl.ds(0, N)])
    pltpu.sync_copy(m_hbm.at[wid], m_v)

    inf16 = jnp.full((16,), jnp.inf, jnp.float32)

    def pad_body(i, c):
        gs_v[pl.ds(N + i * 16, 16)] = inf16
        return c

    lax.fori_loop(0, (NPAD - N) // 16, pad_body, 0)

    one16 = jnp.full((16,), 1.0, jnp.float32)
    neg16 = jnp.full((16,), -1.0, jnp.float32)

    def vsort_pass(kv):
        def body(v, c):
            vec = gs_v[pl.ds(v * 16, 16)]
            asc = (v & kv) == 0
            s = jnp.where(asc, one16, neg16)
            gs_v[pl.ds(v * 16, 16)] = _sorted16(vec * s) * s
            return c

        lax.fori_loop(0, NVREG, body, 0)

    vsort_pass(1)
    for t in range(6):
        kv = 2 << t
        for u in range(t, -1, -1):
            jv = 1 << u

            def cross_body(p, c, u=u, jv=jv, kv=kv):
                a = ((p >> u) << (u + 1)) | (p & (jv - 1))
                b = a + jv
                asc = (a & kv) == 0
                va = gs_v[pl.ds(a * 16, 16)]
                vb = gs_v[pl.ds(b * 16, 16)]
                lo = jnp.minimum(va, vb)
                hi = jnp.maximum(va, vb)
                gs_v[pl.ds(a * 16, 16)] = jnp.where(asc, lo, hi)
                gs_v[pl.ds(b * 16, 16)] = jnp.where(asc, hi, lo)
                return c

            lax.fori_loop(0, NVREG // 2, cross_body, 0)
        vsort_pass(kv)

    def mmx_body(j, carry):
        mn, mx = carry
        mvec = m_v[pl.ds(j * 16, 16)]
        return jnp.minimum(mn, mvec), jnp.maximum(mx, mvec)

    m_mn, m_mx = lax.fori_loop(0, NJ, mmx_body, (inf16, -inf16))
    m_mn = jnp.full((16,), jnp.min(m_mn), jnp.float32)
    m_mx = jnp.full((16,), jnp.max(m_mx), jnp.float32)
    g_mn = jnp.full((16,), jnp.min(gs_v[pl.ds(0, 16)]), jnp.float32)
    g_mx = jnp.full((16,), jnp.max(gs_v[pl.ds((N // 16 - 1) * 16, 16)]),
                    jnp.float32)
    p1, p2 = g_mn * m_mn, g_mn * m_mx
    p3, p4 = g_mx * m_mn, g_mx * m_mx
    pmin = jnp.minimum(jnp.minimum(p1, p2), jnp.minimum(p3, p4))
    pmax = jnp.maximum(jnp.maximum(p1, p2), jnp.maximum(p3, p4))
    lo_init = jnp.min(_keys31_v(pmin)) - 1
    hi_init = jnp.max(_keys31_v(pmax))

    zero16 = jnp.zeros((16,), jnp.int32)
    n16 = jnp.full((16,), N, jnp.int32)
    int_min16 = jnp.full((16,), INT_MIN32, jnp.int32)

    def count_quad(j4, tot, v):
        cnts = []
        for q in range(4):
            mvec = m_v[pl.ds((j4 * 4 + q) * 16, 16)]
            neg = mvec < 0.0
            loi = zero16
            hii = n16
            for _ in range(10):
                midi = lax.shift_right_arithmetic(loi + hii, 1)
                gv = plsc.load_gather(gs_v, [midi])
                le = (gv * mvec) <= v
                pr = le != neg
                loi = jnp.where(pr, midi + 1, loi)
                hii = jnp.where(pr, hii, midi)
            cnts.append(jnp.where(neg, n16 - loi, loi))
        return tot + jnp.sum(cnts[0] + cnts[1] + cnts[2] + cnts[3])

    def titer(_, carry):
        lo_k, hi_k = carry
        mid = lax.shift_right_arithmetic(lo_k + hi_k, 1)
        midv = jnp.full((16,), mid, jnp.int32)
        v = _decode31_hi(midv, int_min16, jnp.full((16,), jnp.inf,
                                                   jnp.float32))
        c = lax.fori_loop(0, NJ // 4, lambda j4, tot: count_quad(j4, tot, v),
                          jnp.int32(0))
        pred = c >= K_ZERO + 1
        return (jnp.where(pred, lo_k, mid), jnp.where(pred, mid, hi_k))

    _, thr = lax.fori_loop(0, 32, titer, (lo_init, hi_init))

    thrv = jnp.full((16,), thr, jnp.int32)
    c0 = lax.shift_left(thrv, 1)
    c1 = c0 | 1
    f0 = plsc.bitcast(jnp.where(c0 >= 0, c0, int_min16 - c0), jnp.float32)
    f1 = plsc.bitcast(jnp.where(c1 >= 0, c1, int_min16 - c1), jnp.float32)
    out_v[...] = jnp.minimum(f0, f1)
    pltpu.sync_copy(out_v, thr_hbm.at[wid])


def _thresholds_sc(g2, m2):
    mesh = plsc.VectorSubcoreMesh(
        core_axis_name="c", subcore_axis_name="s", num_cores=2,
        num_subcores=16)
    return pl.kernel(
        _sc_body,
        out_type=jax.ShapeDtypeStruct((BW, 16), jnp.float32),
        mesh=mesh,
        scratch_types=[
            pltpu.VMEM((NPAD,), jnp.float32),
            pltpu.VMEM((N,), jnp.float32),
            pltpu.VMEM((16,), jnp.float32),
        ],
        compiler_params=pltpu.CompilerParams(
            needs_layout_passes=False, use_tc_tiling_on_sc=False),
    )(g2, m2)


def kernel(x):
    b, w, n, h = x.shape
    xr = x.reshape(b * w, n, h)
    g3, m3 = pl.pallas_call(
        _stage_a_body,
        grid=(BW,),
        in_specs=[pl.BlockSpec((1, N, H), lambda i: (i, 0, 0))],
        out_specs=[
            pl.BlockSpec((1, 1, N), lambda i: (i, 0, 0)),
            pl.BlockSpec((1, 1, N), lambda i: (i, 0, 0)),
        ],
        out_shape=[
            jax.ShapeDtypeStruct((BW, 1, N), jnp.float32),
            jax.ShapeDtypeStruct((BW, 1, N), jnp.float32),
        ],
    )(xr)
    wthr = _thresholds_sc(g3.reshape(BW, N), m3.reshape(BW, N))
    dmap = pl.pallas_call(
        _stage_b_body,
        grid=(BW,),
        in_specs=[
            pl.BlockSpec((1, 1, N), lambda i: (i, 0, 0)),
            pl.BlockSpec((1, 1, N), lambda i: (i, 0, 0)),
            pl.BlockSpec((1, 1, 16), lambda i: (i, 0, 0),
                         memory_space=pltpu.SMEM),
        ],
        out_specs=pl.BlockSpec((1, N, N), lambda i: (i, 0, 0)),
        out_shape=jax.ShapeDtypeStruct((BW, N, N), jnp.float32),
    )(g3, m3, wthr.reshape(BW, 1, 16))
    return xr, dmap
